# baseline (speedup 1.0000x reference)
import functools

import jax
import jax.numpy as jnp
from jax import lax
from jax.experimental import pallas as pl
from jax.experimental.pallas import tpu as pltpu
from jax.experimental.pallas import tpu_sc as plsc

VOCAB = 1000
BATCH = 4096
SEQ = 20

NC = 2
NS = 16
V_STRIPE = 64
V_LAST = VOCAB - (NS - 1) * V_STRIPE
B_HALF = BATCH // NC
NB = B_HALF // 128
BG = 128 // 16


def _sc_gather_t(table, idx_flat):
    mesh = plsc.VectorSubcoreMesh(core_axis_name="c", subcore_axis_name="s")

    @functools.partial(
        pl.kernel,
        mesh=mesh,
        out_type=jax.ShapeDtypeStruct(
            (SEQ, VOCAB // 8, BATCH // 128, 8, 128), jnp.float32
        ),
        scratch_types=[
            pltpu.VMEM((VOCAB, V_STRIPE), jnp.float32),
            pltpu.VMEM((SEQ * B_HALF,), jnp.int32),
            pltpu.VMEM((V_STRIPE // 8, 8, 128), jnp.float32),
            pltpu.VMEM((V_STRIPE // 8, 8, 128), jnp.float32),
            pltpu.SemaphoreType.DMA,
            pltpu.SemaphoreType.DMA,
        ],
        compiler_params=pltpu.CompilerParams(
            use_tc_tiling_on_sc=False, needs_layout_passes=False
        ),
    )
    def k(tab_hbm, idx_hbm, out_hbm, stripe, idxs, stg0, stg1, sem0, sem1):
        c = lax.axis_index("c")
        s = lax.axis_index("s")
        cv0 = jnp.where(s == NS - 1, VOCAB - V_STRIPE, s * V_STRIPE)
        v_off = jnp.where(s == NS - 1, V_STRIPE - V_LAST, 0)
        nv = jnp.where(s == NS - 1, V_LAST, V_STRIPE)
        vt0 = s * (V_STRIPE // 8)

        pltpu.sync_copy(tab_hbm.at[pl.ds(0, VOCAB), pl.ds(cv0, V_STRIPE)], stripe)
        pltpu.sync_copy(
            idx_hbm.at[pl.ds(c * SEQ * B_HALF, SEQ * B_HALF)], idxs
        )
        i20 = lax.iota(jnp.int32, 16) * SEQ

        def gather_block(t, bb, stg):
            for bg in range(BG):
                bvec = (bb * 128 + bg * 16) * SEQ + t + i20
                r16 = plsc.load_gather(idxs, [bvec])
                cvec0 = jnp.zeros((16,), jnp.int32) + v_off

                @plsc.parallel_loop(0, nv, unroll=8, carry=cvec0)
                def vloop(v_l, cvec):
                    vals = plsc.load_gather(stripe, [r16, cvec])
                    stg[v_l // 8, v_l % 8, pl.ds(bg * 16, 16)] = vals
                    return cvec + 1

        def issue(t, bb, stg, sem):
            bt = c * NB + bb

            @pl.when(s != NS - 1)
            def _():
                pltpu.async_copy(
                    stg,
                    out_hbm.at[t, pl.ds(vt0, V_STRIPE // 8), bt],
                    sem,
                )

            @pl.when(s == NS - 1)
            def _():
                def body(vt, carry):
                    pltpu.async_copy(
                        stg.at[pl.ds(vt, 1)],
                        out_hbm.at[t, pl.ds(vt0 + vt, 1), bt],
                        sem,
                    )
                    return carry

                lax.fori_loop(0, V_LAST // 8, body, 0)

        def wait_prev(stg, sem):
            @pl.when(s != NS - 1)
            def _():
                pltpu.make_async_copy(
                    out_hbm.at[0, pl.ds(0, V_STRIPE // 8), 0], stg, sem
                ).wait()

            @pl.when(s == NS - 1)
            def _():
                def body(vt, carry):
                    pltpu.make_async_copy(
                        out_hbm.at[0, pl.ds(0, 1), 0],
                        stg.at[pl.ds(0, 1)],
                        sem,
                    ).wait()
                    return carry

                lax.fori_loop(0, V_LAST // 8, body, 0)

        def t_body(t, carry):
            def pp_body(pp, carry2):
                for par, (stg, sem) in enumerate(((stg0, sem0), (stg1, sem1))):
                    bb = pp * 2 + par
                    not_first = jnp.logical_or(t > 0, pp > 0)

                    @pl.when(not_first)
                    def _():
                        wait_prev(stg, sem)

                    gather_block(t, bb, stg)
                    issue(t, bb, stg, sem)
                return carry2

            return lax.fori_loop(0, NB // 2, pp_body, carry)

        lax.fori_loop(0, SEQ, t_body, 0)
        wait_prev(stg0, sem0)
        wait_prev(stg1, sem1)

    return k(table, idx_flat)


def kernel(idx, table):
    out5 = _sc_gather_t(table, idx.reshape(-1).astype(jnp.int32))
    out_t = out5.transpose(0, 1, 3, 2, 4).reshape(SEQ, VOCAB, BATCH)
    return jnp.transpose(out_t, (2, 0, 1))


# speedup vs baseline: 4.1328x; 4.1328x over previous
import functools

import jax
import jax.numpy as jnp
from jax import lax
from jax.experimental import pallas as pl
from jax.experimental.pallas import tpu as pltpu
from jax.experimental.pallas import tpu_sc as plsc

VOCAB = 1000
BATCH = 4096
SEQ = 20

NC = 2
NS = 16
V_STRIPE = 64
V_LAST = VOCAB - (NS - 1) * V_STRIPE
B_HALF = BATCH // NC
NB = B_HALF // 128
BG = 128 // 16


def _sc_gather_t(table_t_flat, idx_a):
    mesh = plsc.VectorSubcoreMesh(core_axis_name="c", subcore_axis_name="s")

    @functools.partial(
        pl.kernel,
        mesh=mesh,
        out_type=jax.ShapeDtypeStruct(
            (SEQ, VOCAB // 8, BATCH // 128, 8, 128), jnp.float32
        ),
        scratch_types=[
            pltpu.VMEM((V_STRIPE * VOCAB,), jnp.float32),
            pltpu.VMEM((SEQ * B_HALF,), jnp.int32),
            pltpu.VMEM((V_STRIPE // 8, 8, 128), jnp.float32),
            pltpu.VMEM((V_STRIPE // 8, 8, 128), jnp.float32),
            pltpu.SemaphoreType.DMA,
            pltpu.SemaphoreType.DMA,
        ],
        compiler_params=pltpu.CompilerParams(
            use_tc_tiling_on_sc=False, needs_layout_passes=False
        ),
    )
    def k(tab_hbm, idx_hbm, out_hbm, stripe, idxs, stg0, stg1, sem0, sem1):
        c = lax.axis_index("c")
        s = lax.axis_index("s")
        v0 = s * V_STRIPE
        nv = jnp.where(s == NS - 1, V_LAST, V_STRIPE)

        pltpu.sync_copy(tab_hbm.at[pl.ds(v0 * VOCAB, V_STRIPE * VOCAB)], stripe)
        pltpu.sync_copy(idx_hbm.at[pl.ds(c * SEQ * B_HALF, SEQ * B_HALF)], idxs)

        def gather_block(t, bb, stg):
            base_i = t * B_HALF + bb * 128
            for bg in range(BG):
                r16 = idxs[pl.ds(base_i + bg * 16, 16)]

                @plsc.parallel_loop(0, nv, unroll=16, carry=r16)
                def vloop(v_l, gidx):
                    vals = plsc.load_gather(stripe, [gidx])
                    stg[v_l // 8, v_l % 8, pl.ds(bg * 16, 16)] = vals
                    return gidx + VOCAB

        vt0 = s * (V_STRIPE // 8)

        def issue(t, bb, stg, sem):
            bt = c * NB + bb

            @pl.when(s != NS - 1)
            def _():
                pltpu.async_copy(
                    stg,
                    out_hbm.at[t, pl.ds(vt0, V_STRIPE // 8), bt],
                    sem,
                )

            @pl.when(s == NS - 1)
            def _():
                def body(vt, carry):
                    pltpu.async_copy(
                        stg.at[pl.ds(vt, 1)],
                        out_hbm.at[t, pl.ds(vt0 + vt, 1), bt],
                        sem,
                    )
                    return carry

                lax.fori_loop(0, V_LAST // 8, body, 0)

        def wait_prev(stg, sem):
            @pl.when(s != NS - 1)
            def _():
                pltpu.make_async_copy(
                    out_hbm.at[0, pl.ds(0, V_STRIPE // 8), 0], stg, sem
                ).wait()

            @pl.when(s == NS - 1)
            def _():
                def body(vt, carry):
                    pltpu.make_async_copy(
                        out_hbm.at[0, pl.ds(0, 1), 0],
                        stg.at[pl.ds(0, 1)],
                        sem,
                    ).wait()
                    return carry

                lax.fori_loop(0, V_LAST // 8, body, 0)

        def t_body(t, carry):
            def pp_body(pp, carry2):
                for par, (stg, sem) in enumerate(((stg0, sem0), (stg1, sem1))):
                    bb = pp * 2 + par
                    not_first = jnp.logical_or(t > 0, pp > 0)

                    @pl.when(not_first)
                    def _():
                        wait_prev(stg, sem)

                    gather_block(t, bb, stg)
                    issue(t, bb, stg, sem)
                return carry2

            return lax.fori_loop(0, NB // 2, pp_body, carry)

        lax.fori_loop(0, SEQ, t_body, 0)
        wait_prev(stg0, sem0)
        wait_prev(stg1, sem1)

    return k(table_t_flat, idx_a)


def kernel(idx, table):
    tab_t = jnp.pad(table.T, ((0, NS * V_STRIPE - VOCAB), (0, 0)))
    tab_t_flat = tab_t.reshape(-1)
    idx_a = (
        idx.astype(jnp.int32).T.reshape(SEQ, NC, B_HALF)
        .swapaxes(0, 1)
        .reshape(-1)
    )
    out5 = _sc_gather_t(tab_t_flat, idx_a)
    out_t = out5.transpose(0, 1, 3, 2, 4).reshape(SEQ, VOCAB, BATCH)
    return jnp.transpose(out_t, (2, 0, 1))


# speedup vs baseline: 5.3485x; 1.2942x over previous
import functools

import jax
import jax.numpy as jnp
from jax import lax
from jax.experimental import pallas as pl
from jax.experimental.pallas import tpu as pltpu
from jax.experimental.pallas import tpu_sc as plsc

VOCAB = 1000
BATCH = 4096
SEQ = 20

NC = 2
NS = 16
V_STRIPE = 64
V_LAST = VOCAB - (NS - 1) * V_STRIPE
B_HALF = BATCH // NC
NB = B_HALF // 128
BG = 128 // 16


def _sc_gather_t(table_t_flat, idx_a):
    mesh = plsc.VectorSubcoreMesh(core_axis_name="c", subcore_axis_name="s")

    @functools.partial(
        pl.kernel,
        mesh=mesh,
        out_type=jax.ShapeDtypeStruct(
            (SEQ, VOCAB // 8, BATCH // 128, 8, 128), jnp.float32
        ),
        scratch_types=[
            pltpu.VMEM((V_STRIPE * VOCAB,), jnp.float32),
            pltpu.VMEM((SEQ * B_HALF,), jnp.int32),
            pltpu.VMEM((V_STRIPE // 8, 8, 128), jnp.float32),
            pltpu.VMEM((V_STRIPE // 8, 8, 128), jnp.float32),
            pltpu.SemaphoreType.DMA,
            pltpu.SemaphoreType.DMA,
        ],
        compiler_params=pltpu.CompilerParams(
            use_tc_tiling_on_sc=False, needs_layout_passes=False
        ),
    )
    def k(tab_hbm, idx_hbm, out_hbm, stripe, idxs, stg0, stg1, sem0, sem1):
        c = lax.axis_index("c")
        s = lax.axis_index("s")
        v0 = s * V_STRIPE
        nv = jnp.where(s == NS - 1, V_LAST, V_STRIPE)

        pltpu.sync_copy(tab_hbm.at[pl.ds(v0 * VOCAB, V_STRIPE * VOCAB)], stripe)
        pltpu.sync_copy(idx_hbm.at[pl.ds(c * SEQ * B_HALF, SEQ * B_HALF)], idxs)

        def gather_block(t, bb, stg):
            base_i = t * B_HALF + bb * 128
            for bg in range(BG):
                r16 = idxs[pl.ds(base_i + bg * 16, 16)]

                @plsc.parallel_loop(0, nv, unroll=8, carry=r16)
                def vloop(v_l, gidx):
                    vals = plsc.load_gather(stripe, [gidx])
                    stg[v_l // 8, v_l % 8, pl.ds(bg * 16, 16)] = vals
                    return gidx + VOCAB

        vt0 = s * (V_STRIPE // 8)

        def issue(t, bb, stg, sem):
            bt = c * NB + bb

            @pl.when(s != NS - 1)
            def _():
                pltpu.async_copy(
                    stg,
                    out_hbm.at[t, pl.ds(vt0, V_STRIPE // 8), bt],
                    sem,
                )

            @pl.when(s == NS - 1)
            def _():
                def body(vt, carry):
                    pltpu.async_copy(
                        stg.at[pl.ds(vt, 1)],
                        out_hbm.at[t, pl.ds(vt0 + vt, 1), bt],
                        sem,
                    )
                    return carry

                lax.fori_loop(0, V_LAST // 8, body, 0)

        def wait_prev(stg, sem):
            @pl.when(s != NS - 1)
            def _():
                pltpu.make_async_copy(
                    out_hbm.at[0, pl.ds(0, V_STRIPE // 8), 0], stg, sem
                ).wait()

            @pl.when(s == NS - 1)
            def _():
                def body(vt, carry):
                    pltpu.make_async_copy(
                        out_hbm.at[0, pl.ds(0, 1), 0],
                        stg.at[pl.ds(0, 1)],
                        sem,
                    ).wait()
                    return carry

                lax.fori_loop(0, V_LAST // 8, body, 0)

        def t_body(t, carry):
            def pp_body(pp, carry2):
                for par, (stg, sem) in enumerate(((stg0, sem0), (stg1, sem1))):
                    bb = pp * 2 + par
                    not_first = jnp.logical_or(t > 0, pp > 0)

                    @pl.when(not_first)
                    def _():
                        wait_prev(stg, sem)

                    gather_block(t, bb, stg)
                    issue(t, bb, stg, sem)
                return carry2

            return lax.fori_loop(0, NB // 2, pp_body, carry)

        lax.fori_loop(0, SEQ, t_body, 0)
        wait_prev(stg0, sem0)
        wait_prev(stg1, sem1)

    return k(table_t_flat, idx_a)


def kernel(idx, table):
    tab_t = jnp.pad(table.T, ((0, NS * V_STRIPE - VOCAB), (0, 0)))
    tab_t_flat = tab_t.reshape(-1)
    idx_a = (
        idx.astype(jnp.int32).T.reshape(SEQ, NC, B_HALF)
        .swapaxes(0, 1)
        .reshape(-1)
    )
    out5 = _sc_gather_t(tab_t_flat, idx_a)
    out_t = out5.transpose(0, 1, 3, 2, 4).reshape(SEQ, VOCAB, BATCH)
    return jnp.transpose(out_t, (2, 0, 1))
